# baseline (speedup 1.0000x reference)
import functools

import jax
import jax.numpy as jnp
from jax import lax
from jax.experimental import pallas as pl
from jax.experimental.pallas import tpu as pltpu
from jax.experimental.pallas import tpu_sc as plsc

_LEVELS = [10242, 2562, 642, 162, 42]
_NW = 32


def _pad8(n):
    return ((n + 7) // 8) * 8


def _gather_plan(C, M):
    if C == 1:
        return 16, 128
    cw = (230 * 1024) // (C * 4)
    cw = max(8, min(1024, cw - cw % 8))
    cw = min(cw, M)
    cmin = min(cw, 128)
    kc = min(cw // cmin, 8)
    return kc, cmin


def _gather_body(M, cw, kc, cmin, nl):
    def gk(table_hbm, idx_hbm, out_hbm, idx_v0, idx_v1, rows_v0, rows_v1,
           isems, gsems, ssems):
        wid = lax.axis_index("s") * 2 + lax.axis_index("c")
        idx_b = (idx_v0, idx_v1)
        rows_b = (rows_v0, rows_v1)

        def off(it):
            return jnp.minimum((wid * nl + it) * cw, M - cw)

        def idx_cp(it):
            b = it % 2
            return pltpu.make_async_copy(
                idx_hbm.at[pl.ds(off(it), cw)], idx_b[b], isems.at[b]
            )

        def gath_cps(it):
            b = it % 2
            cps = []
            for j in range(kc):
                sl = pl.ds(j * cmin, cmin)
                cps.append(
                    pltpu.make_async_copy(
                        table_hbm.at[idx_b[b].at[sl]],
                        rows_b[b].at[sl],
                        gsems.at[b],
                    )
                )
            return cps

        def store_cp(it):
            b = it % 2
            return pltpu.make_async_copy(
                rows_b[b], out_hbm.at[pl.ds(off(it), cw)], ssems.at[b]
            )

        idx_cp(0).start()
        if nl > 1:
            idx_cp(1).start()
        for it in range(nl):
            if it >= 2:
                store_cp(it - 2).wait()
            idx_cp(it).wait()
            cps = gath_cps(it)
            for cp in cps:
                cp.start()
            for cp in cps:
                cp.wait()
            if it + 2 < nl:
                idx_cp(it + 2).start()
            store_cp(it).start()
        if nl > 1:
            store_cp(nl - 2).wait()
        store_cp(nl - 1).wait()

    return gk


@functools.cache
def _gather_call(V, C, M):
    kc, cmin = _gather_plan(C, M)
    cw = kc * cmin
    nl = -(-(-(-M // cw)) // _NW)
    flat = C == 1
    mesh = plsc.VectorSubcoreMesh(core_axis_name="c", subcore_axis_name="s")
    return functools.partial(
        pl.kernel,
        mesh=mesh,
        out_type=jax.ShapeDtypeStruct((M,) if flat else (M, C), jnp.float32),
        scratch_types=[
            pltpu.VMEM((cw,), jnp.int32),
            pltpu.VMEM((cw,), jnp.int32),
            pltpu.VMEM((cw,) if flat else (cw, C), jnp.float32),
            pltpu.VMEM((cw,) if flat else (cw, C), jnp.float32),
            pltpu.SemaphoreType.DMA((2,)),
            pltpu.SemaphoreType.DMA((2,)),
            pltpu.SemaphoreType.DMA((2,)),
        ],
    )(_gather_body(M, cw, kc, cmin, nl))


@functools.cache
def _oh_gather_call(V, C, M):
    def body(idx_ref, t_ref, o_ref):
        idx = idx_ref[0, 0]
        oh = (
            idx[:, None] == lax.broadcasted_iota(jnp.int32, (1, V), 1)
        ).astype(jnp.float32)
        o_ref[...] = jnp.dot(oh, t_ref[...], preferred_element_type=jnp.float32)

    return pl.pallas_call(
        body, out_shape=jax.ShapeDtypeStruct((M, C), jnp.float32)
    )


def _use_oh(V, C, M):
    return V <= 1280 and 2 * M * V * C <= 2.0e9


def _gather(table, idx, M):
    V, C = table.shape
    if _use_oh(V, C, M):
        return _oh_gather_call(V, C, M)(idx.reshape(1, 1, M), table)
    return _gather_call(V, C, M)(table, idx)


@functools.cache
def _mmk_fused_call(R, Mp, C, F, bm):
    gm = -(-Mp // bm)

    def body(x_ref, w_ref, b_ref, o_ref):
        acc = (
            jnp.dot(x_ref[0], w_ref[0], preferred_element_type=jnp.float32)
            + b_ref[...]
        )
        for k in range(1, R):
            acc += jnp.dot(
                x_ref[k], w_ref[k], preferred_element_type=jnp.float32
            )
        o_ref[...] = acc

    return pl.pallas_call(
        body,
        grid=(gm,),
        in_specs=[
            pl.BlockSpec((R, bm, C), lambda i: (0, i, 0)),
            pl.BlockSpec((R, C, F), lambda i: (0, 0, 0)),
            pl.BlockSpec((1, F), lambda i: (0, 0)),
        ],
        out_specs=pl.BlockSpec((bm, F), lambda i: (i, 0)),
        out_shape=jax.ShapeDtypeStruct((Mp, F), jnp.float32),
        compiler_params=pltpu.CompilerParams(dimension_semantics=("parallel",)),
    )


def _mmk(x3, w3, b2):
    R, Mp, C = x3.shape
    F = w3.shape[2]
    if R * C * F * 4 <= 10 * 2**20:
        bm = min(Mp, 512, ((2**23 // (R * C * 4)) // 8) * 8)
        bm = max(bm, 8)
        return _mmk_fused_call(R, Mp, C, F, bm)(x3, w3, b2)
    return _mmk_call(R, Mp, C, F, min(Mp, 512))(x3, w3, b2)


@functools.cache
def _mmk_call(R, Mp, C, F, bm):
    gm = -(-Mp // bm)

    def body(x_ref, w_ref, b_ref, o_ref):
        acc = jnp.dot(x_ref[0], w_ref[0], preferred_element_type=jnp.float32)

        @pl.when(pl.program_id(1) == 0)
        def _():
            o_ref[...] = acc + b_ref[...]

        @pl.when(pl.program_id(1) != 0)
        def _():
            o_ref[...] += acc

    return pl.pallas_call(
        body,
        grid=(gm, R),
        in_specs=[
            pl.BlockSpec((1, bm, C), lambda i, k: (k, i, 0)),
            pl.BlockSpec((1, C, F), lambda i, k: (k, 0, 0)),
            pl.BlockSpec((1, F), lambda i, k: (0, 0)),
        ],
        out_specs=pl.BlockSpec((bm, F), lambda i, k: (i, 0)),
        out_shape=jax.ShapeDtypeStruct((Mp, F), jnp.float32),
        compiler_params=pltpu.CompilerParams(
            dimension_semantics=("parallel", "arbitrary")
        ),
    )


@functools.cache
def _mm_call(M, K, F, bm, bk):
    gm = -(-M // bm)
    gk_ = K // bk

    def body(x_ref, w_ref, b_ref, o_ref):
        acc = jnp.dot(x_ref[...], w_ref[...], preferred_element_type=jnp.float32)

        @pl.when(pl.program_id(1) == 0)
        def _():
            o_ref[...] = acc + b_ref[...]

        @pl.when(pl.program_id(1) != 0)
        def _():
            o_ref[...] += acc

    return pl.pallas_call(
        body,
        grid=(gm, gk_),
        in_specs=[
            pl.BlockSpec((bm, bk), lambda i, k: (i, k)),
            pl.BlockSpec((bk, F), lambda i, k: (k, 0)),
            pl.BlockSpec((1, F), lambda i, k: (0, 0)),
        ],
        out_specs=pl.BlockSpec((bm, F), lambda i, k: (i, 0)),
        out_shape=jax.ShapeDtypeStruct((M, F), jnp.float32),
        compiler_params=pltpu.CompilerParams(
            dimension_semantics=("parallel", "arbitrary")
        ),
    )


def _mm(x, w, b):
    M, K = x.shape
    F = w.shape[1]
    bm = min(M, 512)
    bk = 2432 if (K % 2432 == 0 and K > 2432) else K
    return _mm_call(M, K, F, bm, bk)(x, w, b.reshape(1, F))


@functools.cache
def _mmup_call(Mp, K, Fp, bm):
    gm = -(-Mp // bm)

    def body(x_ref, w_ref, b_ref, o_ref):
        o_ref[0] = (
            jnp.dot(x_ref[...], w_ref[0], preferred_element_type=jnp.float32)
            + b_ref[0]
        )

    return pl.pallas_call(
        body,
        grid=(gm, 7),
        in_specs=[
            pl.BlockSpec((bm, K), lambda i, j: (i, 0)),
            pl.BlockSpec((1, K, Fp), lambda i, j: (j, 0, 0)),
            pl.BlockSpec((1, 1, Fp), lambda i, j: (j, 0, 0)),
        ],
        out_specs=pl.BlockSpec((1, bm, Fp), lambda i, j: (j, i, 0)),
        out_shape=jax.ShapeDtypeStruct((7, Mp, Fp), jnp.float32),
        compiler_params=pltpu.CompilerParams(
            dimension_semantics=("parallel", "arbitrary")
        ),
    )


@functools.cache
def _mmt_call(Mp, bm):
    gm = -(-Mp // bm)

    def body(g_ref, w_ref, b_ref, o_ref):
        o_ref[...] = (
            lax.dot_general(
                g_ref[...],
                w_ref[...],
                (((0,), (0,)), ((), ())),
                preferred_element_type=jnp.float32,
            )
            + b_ref[...]
        )

    return pl.pallas_call(
        body,
        grid=(gm,),
        in_specs=[
            pl.BlockSpec((57, bm), lambda i: (0, i)),
            pl.BlockSpec((57, 128), lambda i: (0, 0)),
            pl.BlockSpec((1, 128), lambda i: (0, 0)),
        ],
        out_specs=pl.BlockSpec((bm, 128), lambda i: (i, 0)),
        out_shape=jax.ShapeDtypeStruct((Mp, 128), jnp.float32),
        compiler_params=pltpu.CompilerParams(dimension_semantics=("parallel",)),
    )


@functools.cache
def _bn2_call(Mp, n, F):
    inv_n = 1.0 / n

    def body(ya_ref, yb_ref, g_ref, be_ref, o_ref):
        y = ya_ref[...] + yb_ref[...]
        msk = lax.broadcasted_iota(jnp.int32, (Mp, 1), 0) < n
        ym = jnp.where(msk, y, 0.0)
        mu = jnp.sum(ym, axis=0, keepdims=True) * inv_n
        d = jnp.where(msk, y - mu, 0.0)
        var = jnp.sum(d * d, axis=0, keepdims=True) * inv_n
        h = (y - mu) * lax.rsqrt(var + 1e-5) * g_ref[...] + be_ref[...]
        o_ref[...] = jnp.where(h > 0, h, 0.2 * h)

    return pl.pallas_call(
        body, out_shape=jax.ShapeDtypeStruct((Mp, F), jnp.float32)
    )


@functools.cache
def _bn_call(Mp, n, F):
    inv_n = 1.0 / n

    def body(y_ref, g_ref, be_ref, o_ref):
        y = y_ref[...]
        msk = lax.broadcasted_iota(jnp.int32, (Mp, 1), 0) < n
        ym = jnp.where(msk, y, 0.0)
        mu = jnp.sum(ym, axis=0, keepdims=True) * inv_n
        d = jnp.where(msk, y - mu, 0.0)
        var = jnp.sum(d * d, axis=0, keepdims=True) * inv_n
        h = (y - mu) * lax.rsqrt(var + 1e-5) * g_ref[...] + be_ref[...]
        o_ref[...] = jnp.where(h > 0, h, 0.2 * h)

    return pl.pallas_call(
        body, out_shape=jax.ShapeDtypeStruct((Mp, F), jnp.float32)
    )


@functools.cache
def _mean0_call(R, Q, C):
    def body(x_ref, o_ref):
        o_ref[...] = jnp.mean(x_ref[...], axis=0)

    return pl.pallas_call(
        body, out_shape=jax.ShapeDtypeStruct((Q, C), jnp.float32)
    )


def _kmajor(ne, n, n_pad, R):
    a = ne.astype(jnp.int32).reshape(n, R)
    a = jnp.pad(a, ((0, n_pad - n), (0, 0)))
    return jnp.transpose(a).reshape(R * n_pad)


def _expand_cin(w, cin, cin_p):
    F = w.shape[1]
    w3 = w.reshape(19, cin, F)
    return jnp.pad(w3, ((0, 0), (0, cin_p - cin), (0, 0))).reshape(19 * cin_p, F)


def _conv_bn(h, ne, n, w, b, g, be):
    n_pad, C = h.shape
    cin = w.shape[0] // 19
    if cin != C:
        w = _expand_cin(w, cin, C)
    F = w.shape[1]
    if F < 128:
        w = jnp.pad(w, ((0, 0), (0, 128 - F)))
        b = jnp.pad(b, (0, 128 - F))
        g = jnp.pad(g, (0, 128 - F))
        be = jnp.pad(be, (0, 128 - F))
        F = 128
    idx = _kmajor(ne, n, n_pad, 19)
    w3 = w.reshape(19, C, F)
    bm = min(n_pad, 512)
    g2 = g.reshape(1, F)
    be2 = be.reshape(1, F)
    use_sc = not _use_oh(n_pad, C, 19 * n_pad)
    if use_sc and n_pad >= 640:
        a = 12
        ga = _gather_call(n_pad, C, a * n_pad)(h, idx[: a * n_pad])
        gb = _gather(h, idx[a * n_pad:], (19 - a) * n_pad)
        zb = jnp.zeros((1, F), jnp.float32)
        ya = _mmk(ga.reshape(a, n_pad, C), w3[:a], b.reshape(1, F))
        yb = _mmk(gb.reshape(19 - a, n_pad, C), w3[a:], zb)
        return _bn2_call(n_pad, n, F)(ya, yb, g2, be2)
    xg3 = _gather(h, idx, 19 * n_pad).reshape(19, n_pad, C)
    y = _mmk(xg3, w3, b.reshape(1, F))
    return _bn_call(n_pad, n, F)(y, g2, be2)


def kernel(x, params, indices):
    n0 = _LEVELS[0]
    n0p = _pad8(n0)
    acts = []

    xpad = jnp.pad(x, ((0, n0p - n0), (0, 0)))
    xflat = jnp.transpose(xpad).reshape(3 * n0p)
    ne0 = indices["neigh2_10242"]
    idxk = _kmajor(ne0, n0, n0p, 19)
    idx3 = jnp.concatenate([idxk, idxk + n0p, idxk + 2 * n0p])
    g2 = _gather_call(3 * n0p, 1, 57 * n0p)(xflat, idx3).reshape(57, n0p)
    w0 = jnp.transpose(params["d0c1_w"].reshape(19, 3, 64), (1, 0, 2))
    w0 = jnp.pad(w0.reshape(57, 64), ((0, 0), (0, 64)))
    b0 = jnp.pad(params["d0c1_b"], (0, 64))
    y0 = _mmt_call(n0p, 512)(g2, w0, b0.reshape(1, 128))
    h = _bn_call(n0p, n0, 128)(
        y0,
        jnp.pad(params["d0b1_g"], (0, 64)).reshape(1, 128),
        jnp.pad(params["d0b1_be"], (0, 64)).reshape(1, 128),
    )
    h = _conv_bn(h, ne0, n0, params["d0c2_w"], params["d0c2_b"],
                 params["d0b2_g"], params["d0b2_be"])
    acts.append(h)

    for i in range(1, 5):
        n = _LEVELS[i]
        pidx = indices[f"pool_{_LEVELS[i - 1]}"]
        n_pad = _pad8(n)
        C = h.shape[1]
        idx = _kmajor(pidx, n, n_pad, 7)
        g7 = _gather(h, idx, 7 * n_pad).reshape(7, n_pad, C)
        h = _mean0_call(7, n_pad, C)(g7)
        ne = indices[f"neigh2_{n}"]
        h = _conv_bn(h, ne, n, params[f"d{i}c1_w"], params[f"d{i}c1_b"],
                     params[f"d{i}b1_g"], params[f"d{i}b1_be"])
        h = _conv_bn(h, ne, n, params[f"d{i}c2_w"], params[f"d{i}c2_b"],
                     params[f"d{i}b2_g"], params[f"d{i}b2_be"])
        acts.append(h)

    h = acts[-1]
    for i in range(4):
        n_src = _LEVELS[4 - i]
        n_dst = _LEVELS[3 - i]
        n_srcp = _pad8(n_src)
        wup = params[f"u{i}up_w"]
        bup = params[f"u{i}up_b"]
        K = wup.shape[0]
        cout = wup.shape[1] // 7
        coutp = max(cout, 128)
        w3 = wup.reshape(K, 7, cout)
        if cout < 128:
            w3 = jnp.pad(w3, ((0, 0), (0, 0), (0, coutp - cout)))
            bup = jnp.pad(
                bup.reshape(7, cout), ((0, 0), (0, coutp - cout))
            ).reshape(7 * coutp)
        w3 = jnp.transpose(w3, (1, 0, 2))
        b3 = bup.reshape(7, 1, coutp)
        y3 = _mmup_call(n_srcp, K, coutp, min(n_srcp, 512))(h, w3, b3)
        y = y3.reshape(7 * n_srcp, coutp)
        x1 = y3[0, :n_src, :cout]
        q = n_dst - n_src
        dn = indices[f"updown_{n_src}"].astype(jnp.int32)
        dn = (dn % 7) * n_srcp + dn // 7
        dn = jnp.transpose(dn.reshape(q, 2)).reshape(2 * q)
        gd = _gather(y, dn, q * 2).reshape(2, q, coutp)
        x2 = _mean0_call(2, q, coutp)(gd)[:, :cout]
        skip = acts[3 - i][:n_dst, :cout]
        hcat = jnp.concatenate(
            [jnp.concatenate([x1, x2], axis=0), skip], axis=1
        )
        n_pad = _pad8(n_dst)
        h = jnp.pad(hcat, ((0, n_pad - n_dst), (0, 0)))
        ne = indices[f"neigh2_{n_dst}"]
        h = _conv_bn(h, ne, n_dst, params[f"u{i}c1_w"], params[f"u{i}c1_b"],
                     params[f"u{i}b1_g"], params[f"u{i}b1_be"])
        h = _conv_bn(h, ne, n_dst, params[f"u{i}c2_w"], params[f"u{i}c2_b"],
                     params[f"u{i}b2_g"], params[f"u{i}b2_be"])

    wo = jnp.pad(params["outc_w"], ((0, h.shape[1] - 64), (0, 0)))
    out = _mm(h, wo, params["outc_b"])
    return out[:n0]


# speedup vs baseline: 1.0294x; 1.0294x over previous
import functools

import jax
import jax.numpy as jnp
from jax import lax
from jax.experimental import pallas as pl
from jax.experimental.pallas import tpu as pltpu
from jax.experimental.pallas import tpu_sc as plsc

_LEVELS = [10242, 2562, 642, 162, 42]
_NW = 32


def _pad8(n):
    return ((n + 7) // 8) * 8


def _gather_plan(C, M):
    if C == 1:
        return 16, 128
    cw = (230 * 1024) // (C * 4)
    cw = max(8, min(1024, cw - cw % 8))
    cw = min(cw, M)
    cmin = min(cw, 128)
    kc = min(cw // cmin, 8)
    return kc, cmin


def _gather_body(M, cw, kc, cmin, nl):
    def gk(table_hbm, idx_hbm, out_hbm, idx_v0, idx_v1, rows_v0, rows_v1,
           isems, gsems, ssems):
        wid = lax.axis_index("s") * 2 + lax.axis_index("c")
        idx_b = (idx_v0, idx_v1)
        rows_b = (rows_v0, rows_v1)

        def off(it):
            return jnp.minimum((wid * nl + it) * cw, M - cw)

        def idx_cp(it):
            b = it % 2
            return pltpu.make_async_copy(
                idx_hbm.at[pl.ds(off(it), cw)], idx_b[b], isems.at[b]
            )

        def gath_cps(it):
            b = it % 2
            cps = []
            for j in range(kc):
                sl = pl.ds(j * cmin, cmin)
                cps.append(
                    pltpu.make_async_copy(
                        table_hbm.at[idx_b[b].at[sl]],
                        rows_b[b].at[sl],
                        gsems.at[b],
                    )
                )
            return cps

        def store_cp(it):
            b = it % 2
            return pltpu.make_async_copy(
                rows_b[b], out_hbm.at[pl.ds(off(it), cw)], ssems.at[b]
            )

        idx_cp(0).start()
        if nl > 1:
            idx_cp(1).start()
        for it in range(nl):
            if it >= 2:
                store_cp(it - 2).wait()
            idx_cp(it).wait()
            cps = gath_cps(it)
            for cp in cps:
                cp.start()
            for cp in cps:
                cp.wait()
            if it + 2 < nl:
                idx_cp(it + 2).start()
            store_cp(it).start()
        if nl > 1:
            store_cp(nl - 2).wait()
        store_cp(nl - 1).wait()

    return gk


@functools.cache
def _gather_call(V, C, M):
    kc, cmin = _gather_plan(C, M)
    cw = kc * cmin
    nl = -(-(-(-M // cw)) // _NW)
    flat = C == 1
    mesh = plsc.VectorSubcoreMesh(core_axis_name="c", subcore_axis_name="s")
    return functools.partial(
        pl.kernel,
        mesh=mesh,
        out_type=jax.ShapeDtypeStruct((M,) if flat else (M, C), jnp.float32),
        scratch_types=[
            pltpu.VMEM((cw,), jnp.int32),
            pltpu.VMEM((cw,), jnp.int32),
            pltpu.VMEM((cw,) if flat else (cw, C), jnp.float32),
            pltpu.VMEM((cw,) if flat else (cw, C), jnp.float32),
            pltpu.SemaphoreType.DMA((2,)),
            pltpu.SemaphoreType.DMA((2,)),
            pltpu.SemaphoreType.DMA((2,)),
        ],
    )(_gather_body(M, cw, kc, cmin, nl))


@functools.cache
def _oh_gather_call(V, C, M):
    def body(idx_ref, t_ref, o_ref):
        idx = idx_ref[0, 0]
        oh = (
            idx[:, None] == lax.broadcasted_iota(jnp.int32, (1, V), 1)
        ).astype(jnp.float32)
        o_ref[...] = jnp.dot(oh, t_ref[...], preferred_element_type=jnp.float32)

    return pl.pallas_call(
        body, out_shape=jax.ShapeDtypeStruct((M, C), jnp.float32)
    )


def _use_oh(V, C, M):
    return V <= 1280 and 2 * M * V * C <= 1.2e9


def _gather(table, idx, M):
    V, C = table.shape
    if _use_oh(V, C, M):
        return _oh_gather_call(V, C, M)(idx.reshape(1, 1, M), table)
    return _gather_call(V, C, M)(table, idx)


@functools.cache
def _mmk_fused_call(R, Mp, C, F, bm):
    gm = -(-Mp // bm)

    def body(x_ref, w_ref, b_ref, o_ref):
        acc = (
            jnp.dot(x_ref[0], w_ref[0], preferred_element_type=jnp.float32)
            + b_ref[...]
        )
        for k in range(1, R):
            acc += jnp.dot(
                x_ref[k], w_ref[k], preferred_element_type=jnp.float32
            )
        o_ref[...] = acc

    return pl.pallas_call(
        body,
        grid=(gm,),
        in_specs=[
            pl.BlockSpec((R, bm, C), lambda i: (0, i, 0)),
            pl.BlockSpec((R, C, F), lambda i: (0, 0, 0)),
            pl.BlockSpec((1, F), lambda i: (0, 0)),
        ],
        out_specs=pl.BlockSpec((bm, F), lambda i: (i, 0)),
        out_shape=jax.ShapeDtypeStruct((Mp, F), jnp.float32),
        compiler_params=pltpu.CompilerParams(dimension_semantics=("parallel",)),
    )


def _mmk(x3, w3, b2):
    R, Mp, C = x3.shape
    F = w3.shape[2]
    if R * C * F * 4 <= 10 * 2**20:
        bm = min(Mp, 512, ((2**23 // (R * C * 4)) // 8) * 8)
        bm = max(bm, 8)
        return _mmk_fused_call(R, Mp, C, F, bm)(x3, w3, b2)
    return _mmk_call(R, Mp, C, F, min(Mp, 512))(x3, w3, b2)


@functools.cache
def _mmk_call(R, Mp, C, F, bm):
    gm = -(-Mp // bm)

    def body(x_ref, w_ref, b_ref, o_ref):
        acc = jnp.dot(x_ref[0], w_ref[0], preferred_element_type=jnp.float32)

        @pl.when(pl.program_id(1) == 0)
        def _():
            o_ref[...] = acc + b_ref[...]

        @pl.when(pl.program_id(1) != 0)
        def _():
            o_ref[...] += acc

    return pl.pallas_call(
        body,
        grid=(gm, R),
        in_specs=[
            pl.BlockSpec((1, bm, C), lambda i, k: (k, i, 0)),
            pl.BlockSpec((1, C, F), lambda i, k: (k, 0, 0)),
            pl.BlockSpec((1, F), lambda i, k: (0, 0)),
        ],
        out_specs=pl.BlockSpec((bm, F), lambda i, k: (i, 0)),
        out_shape=jax.ShapeDtypeStruct((Mp, F), jnp.float32),
        compiler_params=pltpu.CompilerParams(
            dimension_semantics=("parallel", "arbitrary")
        ),
    )


@functools.cache
def _mm_call(M, K, F, bm, bk):
    gm = -(-M // bm)
    gk_ = K // bk

    def body(x_ref, w_ref, b_ref, o_ref):
        acc = jnp.dot(x_ref[...], w_ref[...], preferred_element_type=jnp.float32)

        @pl.when(pl.program_id(1) == 0)
        def _():
            o_ref[...] = acc + b_ref[...]

        @pl.when(pl.program_id(1) != 0)
        def _():
            o_ref[...] += acc

    return pl.pallas_call(
        body,
        grid=(gm, gk_),
        in_specs=[
            pl.BlockSpec((bm, bk), lambda i, k: (i, k)),
            pl.BlockSpec((bk, F), lambda i, k: (k, 0)),
            pl.BlockSpec((1, F), lambda i, k: (0, 0)),
        ],
        out_specs=pl.BlockSpec((bm, F), lambda i, k: (i, 0)),
        out_shape=jax.ShapeDtypeStruct((M, F), jnp.float32),
        compiler_params=pltpu.CompilerParams(
            dimension_semantics=("parallel", "arbitrary")
        ),
    )


def _mm(x, w, b):
    M, K = x.shape
    F = w.shape[1]
    bm = min(M, 512)
    bk = 2432 if (K % 2432 == 0 and K > 2432) else K
    return _mm_call(M, K, F, bm, bk)(x, w, b.reshape(1, F))


@functools.cache
def _mmup_fused_call(Mp, K, Fp, bm):
    gm = -(-Mp // bm)

    def body(x_ref, w_ref, b_ref, o_ref):
        x = x_ref[...]
        for j in range(7):
            o_ref[j] = (
                jnp.dot(x, w_ref[j], preferred_element_type=jnp.float32)
                + b_ref[j]
            )

    return pl.pallas_call(
        body,
        grid=(gm,),
        in_specs=[
            pl.BlockSpec((bm, K), lambda i: (i, 0)),
            pl.BlockSpec((7, K, Fp), lambda i: (0, 0, 0)),
            pl.BlockSpec((7, 1, Fp), lambda i: (0, 0, 0)),
        ],
        out_specs=pl.BlockSpec((7, bm, Fp), lambda i: (0, i, 0)),
        out_shape=jax.ShapeDtypeStruct((7, Mp, Fp), jnp.float32),
        compiler_params=pltpu.CompilerParams(dimension_semantics=("parallel",)),
    )


@functools.cache
def _mmup_call(Mp, K, Fp, bm):
    gm = -(-Mp // bm)

    def body(x_ref, w_ref, b_ref, o_ref):
        o_ref[0] = (
            jnp.dot(x_ref[...], w_ref[0], preferred_element_type=jnp.float32)
            + b_ref[0]
        )

    return pl.pallas_call(
        body,
        grid=(gm, 7),
        in_specs=[
            pl.BlockSpec((bm, K), lambda i, j: (i, 0)),
            pl.BlockSpec((1, K, Fp), lambda i, j: (j, 0, 0)),
            pl.BlockSpec((1, 1, Fp), lambda i, j: (j, 0, 0)),
        ],
        out_specs=pl.BlockSpec((1, bm, Fp), lambda i, j: (j, i, 0)),
        out_shape=jax.ShapeDtypeStruct((7, Mp, Fp), jnp.float32),
        compiler_params=pltpu.CompilerParams(
            dimension_semantics=("parallel", "arbitrary")
        ),
    )


@functools.cache
def _mmt_call(Mp, bm):
    gm = -(-Mp // bm)

    def body(g_ref, w_ref, b_ref, o_ref):
        o_ref[...] = (
            lax.dot_general(
                g_ref[...],
                w_ref[...],
                (((0,), (0,)), ((), ())),
                preferred_element_type=jnp.float32,
            )
            + b_ref[...]
        )

    return pl.pallas_call(
        body,
        grid=(gm,),
        in_specs=[
            pl.BlockSpec((57, bm), lambda i: (0, i)),
            pl.BlockSpec((57, 128), lambda i: (0, 0)),
            pl.BlockSpec((1, 128), lambda i: (0, 0)),
        ],
        out_specs=pl.BlockSpec((bm, 128), lambda i: (i, 0)),
        out_shape=jax.ShapeDtypeStruct((Mp, 128), jnp.float32),
        compiler_params=pltpu.CompilerParams(dimension_semantics=("parallel",)),
    )


@functools.cache
def _bn2_call(Mp, n, F):
    inv_n = 1.0 / n

    def body(ya_ref, yb_ref, g_ref, be_ref, o_ref):
        y = ya_ref[...] + yb_ref[...]
        msk = lax.broadcasted_iota(jnp.int32, (Mp, 1), 0) < n
        ym = jnp.where(msk, y, 0.0)
        mu = jnp.sum(ym, axis=0, keepdims=True) * inv_n
        d = jnp.where(msk, y - mu, 0.0)
        var = jnp.sum(d * d, axis=0, keepdims=True) * inv_n
        h = (y - mu) * lax.rsqrt(var + 1e-5) * g_ref[...] + be_ref[...]
        o_ref[...] = jnp.where(h > 0, h, 0.2 * h)

    return pl.pallas_call(
        body, out_shape=jax.ShapeDtypeStruct((Mp, F), jnp.float32)
    )


@functools.cache
def _bn_call(Mp, n, F):
    inv_n = 1.0 / n

    def body(y_ref, g_ref, be_ref, o_ref):
        y = y_ref[...]
        msk = lax.broadcasted_iota(jnp.int32, (Mp, 1), 0) < n
        ym = jnp.where(msk, y, 0.0)
        mu = jnp.sum(ym, axis=0, keepdims=True) * inv_n
        d = jnp.where(msk, y - mu, 0.0)
        var = jnp.sum(d * d, axis=0, keepdims=True) * inv_n
        h = (y - mu) * lax.rsqrt(var + 1e-5) * g_ref[...] + be_ref[...]
        o_ref[...] = jnp.where(h > 0, h, 0.2 * h)

    return pl.pallas_call(
        body, out_shape=jax.ShapeDtypeStruct((Mp, F), jnp.float32)
    )


@functools.cache
def _mean0_call(R, Q, C):
    def body(x_ref, o_ref):
        o_ref[...] = jnp.mean(x_ref[...], axis=0)

    return pl.pallas_call(
        body, out_shape=jax.ShapeDtypeStruct((Q, C), jnp.float32)
    )


def _kmajor(ne, n, n_pad, R):
    a = ne.astype(jnp.int32).reshape(n, R)
    a = jnp.pad(a, ((0, n_pad - n), (0, 0)))
    return jnp.transpose(a).reshape(R * n_pad)


def _expand_cin(w, cin, cin_p):
    F = w.shape[1]
    w3 = w.reshape(19, cin, F)
    return jnp.pad(w3, ((0, 0), (0, cin_p - cin), (0, 0))).reshape(19 * cin_p, F)


def _conv_bn(h, ne, n, w, b, g, be):
    n_pad, C = h.shape
    cin = w.shape[0] // 19
    if cin != C:
        w = _expand_cin(w, cin, C)
    F = w.shape[1]
    if F < 128:
        w = jnp.pad(w, ((0, 0), (0, 128 - F)))
        b = jnp.pad(b, (0, 128 - F))
        g = jnp.pad(g, (0, 128 - F))
        be = jnp.pad(be, (0, 128 - F))
        F = 128
    idx = _kmajor(ne, n, n_pad, 19)
    w3 = w.reshape(19, C, F)
    bm = min(n_pad, 512)
    g2 = g.reshape(1, F)
    be2 = be.reshape(1, F)
    use_sc = not _use_oh(n_pad, C, 19 * n_pad)
    if use_sc and n_pad >= 2048:
        a = 12
        ga = _gather_call(n_pad, C, a * n_pad)(h, idx[: a * n_pad])
        gb = _gather(h, idx[a * n_pad:], (19 - a) * n_pad)
        zb = jnp.zeros((1, F), jnp.float32)
        ya = _mmk(ga.reshape(a, n_pad, C), w3[:a], b.reshape(1, F))
        yb = _mmk(gb.reshape(19 - a, n_pad, C), w3[a:], zb)
        return _bn2_call(n_pad, n, F)(ya, yb, g2, be2)
    xg3 = _gather(h, idx, 19 * n_pad).reshape(19, n_pad, C)
    y = _mmk(xg3, w3, b.reshape(1, F))
    return _bn_call(n_pad, n, F)(y, g2, be2)


def kernel(x, params, indices):
    n0 = _LEVELS[0]
    n0p = _pad8(n0)
    acts = []

    xpad = jnp.pad(x, ((0, n0p - n0), (0, 0)))
    xflat = jnp.transpose(xpad).reshape(3 * n0p)
    ne0 = indices["neigh2_10242"]
    idxk = _kmajor(ne0, n0, n0p, 19)
    idx3 = jnp.concatenate([idxk, idxk + n0p, idxk + 2 * n0p])
    g2 = _gather_call(3 * n0p, 1, 57 * n0p)(xflat, idx3).reshape(57, n0p)
    w0 = jnp.transpose(params["d0c1_w"].reshape(19, 3, 64), (1, 0, 2))
    w0 = jnp.pad(w0.reshape(57, 64), ((0, 0), (0, 64)))
    b0 = jnp.pad(params["d0c1_b"], (0, 64))
    y0 = _mmt_call(n0p, 512)(g2, w0, b0.reshape(1, 128))
    h = _bn_call(n0p, n0, 128)(
        y0,
        jnp.pad(params["d0b1_g"], (0, 64)).reshape(1, 128),
        jnp.pad(params["d0b1_be"], (0, 64)).reshape(1, 128),
    )
    h = _conv_bn(h, ne0, n0, params["d0c2_w"], params["d0c2_b"],
                 params["d0b2_g"], params["d0b2_be"])
    acts.append(h)

    for i in range(1, 5):
        n = _LEVELS[i]
        pidx = indices[f"pool_{_LEVELS[i - 1]}"]
        n_pad = _pad8(n)
        C = h.shape[1]
        idx = _kmajor(pidx, n, n_pad, 7)
        g7 = _gather(h, idx, 7 * n_pad).reshape(7, n_pad, C)
        h = _mean0_call(7, n_pad, C)(g7)
        ne = indices[f"neigh2_{n}"]
        h = _conv_bn(h, ne, n, params[f"d{i}c1_w"], params[f"d{i}c1_b"],
                     params[f"d{i}b1_g"], params[f"d{i}b1_be"])
        h = _conv_bn(h, ne, n, params[f"d{i}c2_w"], params[f"d{i}c2_b"],
                     params[f"d{i}b2_g"], params[f"d{i}b2_be"])
        acts.append(h)

    h = acts[-1]
    for i in range(4):
        n_src = _LEVELS[4 - i]
        n_dst = _LEVELS[3 - i]
        n_srcp = _pad8(n_src)
        wup = params[f"u{i}up_w"]
        bup = params[f"u{i}up_b"]
        K = wup.shape[0]
        cout = wup.shape[1] // 7
        coutp = max(cout, 128)
        w3 = wup.reshape(K, 7, cout)
        if cout < 128:
            w3 = jnp.pad(w3, ((0, 0), (0, 0), (0, coutp - cout)))
            bup = jnp.pad(
                bup.reshape(7, cout), ((0, 0), (0, coutp - cout))
            ).reshape(7 * coutp)
        w3 = jnp.transpose(w3, (1, 0, 2))
        b3 = bup.reshape(7, 1, coutp)
        y3 = _mmup_fused_call(n_srcp, K, coutp, min(n_srcp, 512))(h, w3, b3)
        y = y3.reshape(7 * n_srcp, coutp)
        x1 = y3[0, :n_src, :cout]
        q = n_dst - n_src
        dn = indices[f"updown_{n_src}"].astype(jnp.int32)
        dn = (dn % 7) * n_srcp + dn // 7
        dn = jnp.transpose(dn.reshape(q, 2)).reshape(2 * q)
        gd = _gather(y, dn, q * 2).reshape(2, q, coutp)
        x2 = _mean0_call(2, q, coutp)(gd)[:, :cout]
        skip = acts[3 - i][:n_dst, :cout]
        hcat = jnp.concatenate(
            [jnp.concatenate([x1, x2], axis=0), skip], axis=1
        )
        n_pad = _pad8(n_dst)
        h = jnp.pad(hcat, ((0, n_pad - n_dst), (0, 0)))
        ne = indices[f"neigh2_{n_dst}"]
        h = _conv_bn(h, ne, n_dst, params[f"u{i}c1_w"], params[f"u{i}c1_b"],
                     params[f"u{i}b1_g"], params[f"u{i}b1_be"])
        h = _conv_bn(h, ne, n_dst, params[f"u{i}c2_w"], params[f"u{i}c2_b"],
                     params[f"u{i}b2_g"], params[f"u{i}b2_be"])

    wo = jnp.pad(params["outc_w"], ((0, h.shape[1] - 64), (0, 0)))
    out = _mm(h, wo, params["outc_b"])
    return out[:n0]


# speedup vs baseline: 1.1189x; 1.0870x over previous
import functools

import jax
import jax.numpy as jnp
from jax import lax
from jax.experimental import pallas as pl
from jax.experimental.pallas import tpu as pltpu
from jax.experimental.pallas import tpu_sc as plsc

_LEVELS = [10242, 2562, 642, 162, 42]
_NW = 32


def _pad8(n):
    return ((n + 7) // 8) * 8


def _gather_plan(C, M):
    if C == 1:
        return 16, 128
    cw = (230 * 1024) // (C * 4)
    cw = max(16, min(1024, cw - cw % 16))
    cw = min(cw, M)
    cmin = min(cw, 128)
    kc = min(cw // cmin, 8)
    if kc == 1 and cw >= 16:
        cmin = (cw // 2) - (cw // 2) % 8
        cmin = max(cmin, 8)
        kc = 2
    return kc, cmin


def _gather_body(M, cw, kc, cmin, nl):
    def gk(table_hbm, idx_hbm, out_hbm, idx_v0, idx_v1, rows_v0, rows_v1,
           isems, gsems, ssems):
        wid = lax.axis_index("s") * 2 + lax.axis_index("c")
        idx_b = (idx_v0, idx_v1)
        rows_b = (rows_v0, rows_v1)

        def off(it):
            return jnp.minimum((wid * nl + it) * cw, M - cw)

        def idx_cp(it):
            b = it % 2
            return pltpu.make_async_copy(
                idx_hbm.at[pl.ds(off(it), cw)], idx_b[b], isems.at[b]
            )

        def gath_cps(it):
            b = it % 2
            cps = []
            for j in range(kc):
                sl = pl.ds(j * cmin, cmin)
                cps.append(
                    pltpu.make_async_copy(
                        table_hbm.at[idx_b[b].at[sl]],
                        rows_b[b].at[sl],
                        gsems.at[b],
                    )
                )
            return cps

        def store_cp(it):
            b = it % 2
            return pltpu.make_async_copy(
                rows_b[b], out_hbm.at[pl.ds(off(it), cw)], ssems.at[b]
            )

        idx_cp(0).start()
        if nl > 1:
            idx_cp(1).start()
        for it in range(nl):
            if it >= 2:
                store_cp(it - 2).wait()
            idx_cp(it).wait()
            cps = gath_cps(it)
            for cp in cps:
                cp.start()
            for cp in cps:
                cp.wait()
            if it + 2 < nl:
                idx_cp(it + 2).start()
            store_cp(it).start()
        if nl > 1:
            store_cp(nl - 2).wait()
        store_cp(nl - 1).wait()

    return gk


@functools.cache
def _gather_call(V, C, M):
    kc, cmin = _gather_plan(C, M)
    cw = kc * cmin
    nl = -(-(-(-M // cw)) // _NW)
    flat = C == 1
    mesh = plsc.VectorSubcoreMesh(core_axis_name="c", subcore_axis_name="s")
    return functools.partial(
        pl.kernel,
        mesh=mesh,
        out_type=jax.ShapeDtypeStruct((M,) if flat else (M, C), jnp.float32),
        scratch_types=[
            pltpu.VMEM((cw,), jnp.int32),
            pltpu.VMEM((cw,), jnp.int32),
            pltpu.VMEM((cw,) if flat else (cw, C), jnp.float32),
            pltpu.VMEM((cw,) if flat else (cw, C), jnp.float32),
            pltpu.SemaphoreType.DMA((2,)),
            pltpu.SemaphoreType.DMA((2,)),
            pltpu.SemaphoreType.DMA((2,)),
        ],
    )(_gather_body(M, cw, kc, cmin, nl))


@functools.cache
def _oh_gather_call(V, C, M):
    def body(idx_ref, t_ref, o_ref):
        idx = idx_ref[0, 0]
        oh = (
            idx[:, None] == lax.broadcasted_iota(jnp.int32, (1, V), 1)
        ).astype(jnp.float32)
        o_ref[...] = jnp.dot(oh, t_ref[...], preferred_element_type=jnp.float32)

    return pl.pallas_call(
        body, out_shape=jax.ShapeDtypeStruct((M, C), jnp.float32)
    )


def _use_oh(V, C, M):
    return V <= 1280 and 2 * M * V * C <= 1.2e9


def _gather(table, idx, M):
    V, C = table.shape
    if _use_oh(V, C, M):
        return _oh_gather_call(V, C, M)(idx.reshape(1, 1, M), table)
    return _gather_call(V, C, M)(table, idx)


@functools.cache
def _mmk_fused_call(R, Mp, C, F, bm):
    gm = -(-Mp // bm)

    def body(x_ref, w_ref, b_ref, o_ref):
        acc = (
            jnp.dot(x_ref[0], w_ref[0], preferred_element_type=jnp.float32)
            + b_ref[...]
        )
        for k in range(1, R):
            acc += jnp.dot(
                x_ref[k], w_ref[k], preferred_element_type=jnp.float32
            )
        o_ref[...] = acc

    return pl.pallas_call(
        body,
        grid=(gm,),
        in_specs=[
            pl.BlockSpec((R, bm, C), lambda i: (0, i, 0)),
            pl.BlockSpec((R, C, F), lambda i: (0, 0, 0)),
            pl.BlockSpec((1, F), lambda i: (0, 0)),
        ],
        out_specs=pl.BlockSpec((bm, F), lambda i: (i, 0)),
        out_shape=jax.ShapeDtypeStruct((Mp, F), jnp.float32),
        compiler_params=pltpu.CompilerParams(dimension_semantics=("parallel",)),
    )


def _mmk(x3, w3, b2):
    R, Mp, C = x3.shape
    F = w3.shape[2]
    if R * C * F * 4 <= 10 * 2**20:
        bm = min(Mp, 512, ((2**23 // (R * C * 4)) // 8) * 8)
        bm = max(bm, 8)
        return _mmk_fused_call(R, Mp, C, F, bm)(x3, w3, b2)
    return _mmk_call(R, Mp, C, F, min(Mp, 512))(x3, w3, b2)


@functools.cache
def _mmk_call(R, Mp, C, F, bm):
    gm = -(-Mp // bm)

    def body(x_ref, w_ref, b_ref, o_ref):
        acc = jnp.dot(x_ref[0], w_ref[0], preferred_element_type=jnp.float32)

        @pl.when(pl.program_id(1) == 0)
        def _():
            o_ref[...] = acc + b_ref[...]

        @pl.when(pl.program_id(1) != 0)
        def _():
            o_ref[...] += acc

    return pl.pallas_call(
        body,
        grid=(gm, R),
        in_specs=[
            pl.BlockSpec((1, bm, C), lambda i, k: (k, i, 0)),
            pl.BlockSpec((1, C, F), lambda i, k: (k, 0, 0)),
            pl.BlockSpec((1, F), lambda i, k: (0, 0)),
        ],
        out_specs=pl.BlockSpec((bm, F), lambda i, k: (i, 0)),
        out_shape=jax.ShapeDtypeStruct((Mp, F), jnp.float32),
        compiler_params=pltpu.CompilerParams(
            dimension_semantics=("parallel", "arbitrary")
        ),
    )


@functools.cache
def _mm_call(M, K, F, bm, bk):
    gm = -(-M // bm)
    gk_ = K // bk

    def body(x_ref, w_ref, b_ref, o_ref):
        acc = jnp.dot(x_ref[...], w_ref[...], preferred_element_type=jnp.float32)

        @pl.when(pl.program_id(1) == 0)
        def _():
            o_ref[...] = acc + b_ref[...]

        @pl.when(pl.program_id(1) != 0)
        def _():
            o_ref[...] += acc

    return pl.pallas_call(
        body,
        grid=(gm, gk_),
        in_specs=[
            pl.BlockSpec((bm, bk), lambda i, k: (i, k)),
            pl.BlockSpec((bk, F), lambda i, k: (k, 0)),
            pl.BlockSpec((1, F), lambda i, k: (0, 0)),
        ],
        out_specs=pl.BlockSpec((bm, F), lambda i, k: (i, 0)),
        out_shape=jax.ShapeDtypeStruct((M, F), jnp.float32),
        compiler_params=pltpu.CompilerParams(
            dimension_semantics=("parallel", "arbitrary")
        ),
    )


def _mm(x, w, b):
    M, K = x.shape
    F = w.shape[1]
    bm = min(M, 512)
    bk = 2432 if (K % 2432 == 0 and K > 2432) else K
    return _mm_call(M, K, F, bm, bk)(x, w, b.reshape(1, F))


@functools.cache
def _mmup_fused_call(Mp, K, Fp, bm):
    gm = -(-Mp // bm)

    def body(x_ref, w_ref, b_ref, o_ref):
        x = x_ref[...]
        for j in range(7):
            o_ref[j] = (
                jnp.dot(x, w_ref[j], preferred_element_type=jnp.float32)
                + b_ref[j]
            )

    return pl.pallas_call(
        body,
        grid=(gm,),
        in_specs=[
            pl.BlockSpec((bm, K), lambda i: (i, 0)),
            pl.BlockSpec((7, K, Fp), lambda i: (0, 0, 0)),
            pl.BlockSpec((7, 1, Fp), lambda i: (0, 0, 0)),
        ],
        out_specs=pl.BlockSpec((7, bm, Fp), lambda i: (0, i, 0)),
        out_shape=jax.ShapeDtypeStruct((7, Mp, Fp), jnp.float32),
        compiler_params=pltpu.CompilerParams(dimension_semantics=("parallel",)),
    )


@functools.cache
def _mmup_call(Mp, K, Fp, bm):
    gm = -(-Mp // bm)

    def body(x_ref, w_ref, b_ref, o_ref):
        o_ref[0] = (
            jnp.dot(x_ref[...], w_ref[0], preferred_element_type=jnp.float32)
            + b_ref[0]
        )

    return pl.pallas_call(
        body,
        grid=(gm, 7),
        in_specs=[
            pl.BlockSpec((bm, K), lambda i, j: (i, 0)),
            pl.BlockSpec((1, K, Fp), lambda i, j: (j, 0, 0)),
            pl.BlockSpec((1, 1, Fp), lambda i, j: (j, 0, 0)),
        ],
        out_specs=pl.BlockSpec((1, bm, Fp), lambda i, j: (j, i, 0)),
        out_shape=jax.ShapeDtypeStruct((7, Mp, Fp), jnp.float32),
        compiler_params=pltpu.CompilerParams(
            dimension_semantics=("parallel", "arbitrary")
        ),
    )


@functools.cache
def _mmt_call(Mp, bm):
    gm = -(-Mp // bm)

    def body(g_ref, w_ref, b_ref, o_ref):
        o_ref[...] = (
            lax.dot_general(
                g_ref[...],
                w_ref[...],
                (((0,), (0,)), ((), ())),
                preferred_element_type=jnp.float32,
            )
            + b_ref[...]
        )

    return pl.pallas_call(
        body,
        grid=(gm,),
        in_specs=[
            pl.BlockSpec((57, bm), lambda i: (0, i)),
            pl.BlockSpec((57, 128), lambda i: (0, 0)),
            pl.BlockSpec((1, 128), lambda i: (0, 0)),
        ],
        out_specs=pl.BlockSpec((bm, 128), lambda i: (i, 0)),
        out_shape=jax.ShapeDtypeStruct((Mp, 128), jnp.float32),
        compiler_params=pltpu.CompilerParams(dimension_semantics=("parallel",)),
    )


@functools.cache
def _bn2_call(Mp, n, F):
    inv_n = 1.0 / n

    def body(ya_ref, yb_ref, g_ref, be_ref, o_ref):
        y = ya_ref[...] + yb_ref[...]
        msk = lax.broadcasted_iota(jnp.int32, (Mp, 1), 0) < n
        ym = jnp.where(msk, y, 0.0)
        mu = jnp.sum(ym, axis=0, keepdims=True) * inv_n
        d = jnp.where(msk, y - mu, 0.0)
        var = jnp.sum(d * d, axis=0, keepdims=True) * inv_n
        h = (y - mu) * lax.rsqrt(var + 1e-5) * g_ref[...] + be_ref[...]
        o_ref[...] = jnp.where(h > 0, h, 0.2 * h)

    return pl.pallas_call(
        body, out_shape=jax.ShapeDtypeStruct((Mp, F), jnp.float32)
    )


@functools.cache
def _bn_call(Mp, n, F):
    inv_n = 1.0 / n

    def body(y_ref, g_ref, be_ref, o_ref):
        y = y_ref[...]
        msk = lax.broadcasted_iota(jnp.int32, (Mp, 1), 0) < n
        ym = jnp.where(msk, y, 0.0)
        mu = jnp.sum(ym, axis=0, keepdims=True) * inv_n
        d = jnp.where(msk, y - mu, 0.0)
        var = jnp.sum(d * d, axis=0, keepdims=True) * inv_n
        h = (y - mu) * lax.rsqrt(var + 1e-5) * g_ref[...] + be_ref[...]
        o_ref[...] = jnp.where(h > 0, h, 0.2 * h)

    return pl.pallas_call(
        body, out_shape=jax.ShapeDtypeStruct((Mp, F), jnp.float32)
    )


@functools.cache
def _mean0_call(R, Q, C):
    def body(x_ref, o_ref):
        o_ref[...] = jnp.mean(x_ref[...], axis=0)

    return pl.pallas_call(
        body, out_shape=jax.ShapeDtypeStruct((Q, C), jnp.float32)
    )


def _kmajor(ne, n, n_pad, R):
    a = ne.astype(jnp.int32).reshape(n, R)
    a = jnp.pad(a, ((0, n_pad - n), (0, 0)))
    return jnp.transpose(a).reshape(R * n_pad)


def _expand_cin(w, cin, cin_p):
    F = w.shape[1]
    w3 = w.reshape(19, cin, F)
    return jnp.pad(w3, ((0, 0), (0, cin_p - cin), (0, 0))).reshape(19 * cin_p, F)


def _conv_bn(h, ne, n, w, b, g, be):
    n_pad, C = h.shape
    cin = w.shape[0] // 19
    if cin != C:
        w = _expand_cin(w, cin, C)
    F = w.shape[1]
    if F < 128:
        w = jnp.pad(w, ((0, 0), (0, 128 - F)))
        b = jnp.pad(b, (0, 128 - F))
        g = jnp.pad(g, (0, 128 - F))
        be = jnp.pad(be, (0, 128 - F))
        F = 128
    idx = _kmajor(ne, n, n_pad, 19)
    w3 = w.reshape(19, C, F)
    bm = min(n_pad, 512)
    g2 = g.reshape(1, F)
    be2 = be.reshape(1, F)
    use_sc = not _use_oh(n_pad, C, 19 * n_pad)
    if use_sc and n_pad >= 4096:
        a = 12
        ga = _gather_call(n_pad, C, a * n_pad)(h, idx[: a * n_pad])
        gb = _gather(h, idx[a * n_pad:], (19 - a) * n_pad)
        zb = jnp.zeros((1, F), jnp.float32)
        ya = _mmk(ga.reshape(a, n_pad, C), w3[:a], b.reshape(1, F))
        yb = _mmk(gb.reshape(19 - a, n_pad, C), w3[a:], zb)
        return _bn2_call(n_pad, n, F)(ya, yb, g2, be2)
    xg3 = _gather(h, idx, 19 * n_pad).reshape(19, n_pad, C)
    y = _mmk(xg3, w3, b.reshape(1, F))
    return _bn_call(n_pad, n, F)(y, g2, be2)


def kernel(x, params, indices):
    n0 = _LEVELS[0]
    n0p = _pad8(n0)
    acts = []

    xpad = jnp.pad(x, ((0, n0p - n0), (0, 0)))
    xflat = jnp.transpose(xpad).reshape(3 * n0p)
    ne0 = indices["neigh2_10242"]
    idxk = _kmajor(ne0, n0, n0p, 19)
    idx3 = jnp.concatenate([idxk, idxk + n0p, idxk + 2 * n0p])
    g2 = _gather_call(3 * n0p, 1, 57 * n0p)(xflat, idx3).reshape(57, n0p)
    w0 = jnp.transpose(params["d0c1_w"].reshape(19, 3, 64), (1, 0, 2))
    w0 = jnp.pad(w0.reshape(57, 64), ((0, 0), (0, 64)))
    b0 = jnp.pad(params["d0c1_b"], (0, 64))
    y0 = _mmt_call(n0p, 512)(g2, w0, b0.reshape(1, 128))
    h = _bn_call(n0p, n0, 128)(
        y0,
        jnp.pad(params["d0b1_g"], (0, 64)).reshape(1, 128),
        jnp.pad(params["d0b1_be"], (0, 64)).reshape(1, 128),
    )
    h = _conv_bn(h, ne0, n0, params["d0c2_w"], params["d0c2_b"],
                 params["d0b2_g"], params["d0b2_be"])
    acts.append(h)

    for i in range(1, 5):
        n = _LEVELS[i]
        pidx = indices[f"pool_{_LEVELS[i - 1]}"]
        n_pad = _pad8(n)
        C = h.shape[1]
        idx = _kmajor(pidx, n, n_pad, 7)
        g7 = _gather(h, idx, 7 * n_pad).reshape(7, n_pad, C)
        h = _mean0_call(7, n_pad, C)(g7)
        ne = indices[f"neigh2_{n}"]
        h = _conv_bn(h, ne, n, params[f"d{i}c1_w"], params[f"d{i}c1_b"],
                     params[f"d{i}b1_g"], params[f"d{i}b1_be"])
        h = _conv_bn(h, ne, n, params[f"d{i}c2_w"], params[f"d{i}c2_b"],
                     params[f"d{i}b2_g"], params[f"d{i}b2_be"])
        acts.append(h)

    h = acts[-1]
    for i in range(4):
        n_src = _LEVELS[4 - i]
        n_dst = _LEVELS[3 - i]
        n_srcp = _pad8(n_src)
        wup = params[f"u{i}up_w"]
        bup = params[f"u{i}up_b"]
        K = wup.shape[0]
        cout = wup.shape[1] // 7
        coutp = max(cout, 128)
        w3 = wup.reshape(K, 7, cout)
        if cout < 128:
            w3 = jnp.pad(w3, ((0, 0), (0, 0), (0, coutp - cout)))
            bup = jnp.pad(
                bup.reshape(7, cout), ((0, 0), (0, coutp - cout))
            ).reshape(7 * coutp)
        w3 = jnp.transpose(w3, (1, 0, 2))
        b3 = bup.reshape(7, 1, coutp)
        y3 = _mmup_fused_call(n_srcp, K, coutp, min(n_srcp, 512))(h, w3, b3)
        y = y3.reshape(7 * n_srcp, coutp)
        x1 = y3[0, :n_src, :cout]
        q = n_dst - n_src
        dn = indices[f"updown_{n_src}"].astype(jnp.int32)
        dn = (dn % 7) * n_srcp + dn // 7
        dn = jnp.transpose(dn.reshape(q, 2)).reshape(2 * q)
        gd = _gather(y, dn, q * 2).reshape(2, q, coutp)
        x2 = _mean0_call(2, q, coutp)(gd)[:, :cout]
        skip = acts[3 - i][:n_dst, :cout]
        hcat = jnp.concatenate(
            [jnp.concatenate([x1, x2], axis=0), skip], axis=1
        )
        n_pad = _pad8(n_dst)
        h = jnp.pad(hcat, ((0, n_pad - n_dst), (0, 0)))
        ne = indices[f"neigh2_{n_dst}"]
        h = _conv_bn(h, ne, n_dst, params[f"u{i}c1_w"], params[f"u{i}c1_b"],
                     params[f"u{i}b1_g"], params[f"u{i}b1_be"])
        h = _conv_bn(h, ne, n_dst, params[f"u{i}c2_w"], params[f"u{i}c2_b"],
                     params[f"u{i}b2_g"], params[f"u{i}b2_be"])

    wo = jnp.pad(params["outc_w"], ((0, h.shape[1] - 64), (0, 0)))
    out = _mm(h, wo, params["outc_b"])
    return out[:n0]
